# Initial kernel scaffold; baseline (speedup 1.0000x reference)
import jax
import jax.numpy as jnp
from jax.experimental import pallas as pl


def kernel(token_ids, token_table, pos_table):
    raise NotImplementedError("write your pallas kernel here")


# speedup vs baseline: 2.9805x; 2.9805x over previous
import functools

import jax
import jax.numpy as jnp
from jax import lax
from jax.experimental import pallas as pl
from jax.experimental.pallas import tpu as pltpu
from jax.experimental.pallas import tpu_sc as plsc

D = 128
S = 200
CH = 128
NBUF = 4
NC, NS = 2, 16
NW = NC * NS
NROWS = 4096 * S
NCHUNK = NROWS // CH
CPW = NCHUNK // NW

_mesh = plsc.VectorSubcoreMesh(core_axis_name="c", subcore_axis_name="s")


@functools.partial(
    pl.kernel,
    mesh=_mesh,
    out_type=jax.ShapeDtypeStruct((NROWS, D), jnp.float32),
    scratch_types=[
        pltpu.VMEM((CPW, CH), jnp.int32),
        pltpu.VMEM((S, D), jnp.float32),
        pltpu.VMEM((CH, D), jnp.float32),
        pltpu.VMEM((CH, D), jnp.float32),
        pltpu.VMEM((CH, D), jnp.float32),
        pltpu.VMEM((CH, D), jnp.float32),
        pltpu.SemaphoreType.DMA,
        pltpu.SemaphoreType.DMA,
        pltpu.SemaphoreType.DMA,
        pltpu.SemaphoreType.DMA,
        pltpu.SemaphoreType.DMA,
    ],
)
def _sc_embed(ids_hbm, table_hbm, pos_hbm, out_hbm,
              idx_v, pos_v, r0, r1, r2, r3, gsem, o0, o1, o2, o3):
    rows = (r0, r1, r2, r3)
    osem = (o0, o1, o2, o3)
    wid = lax.axis_index("s") * NC + lax.axis_index("c")
    chunk0 = wid * CPW

    def start_gather(c_local, b):
        pltpu.async_copy(table_hbm.at[idx_v.at[c_local]], rows[b], gsem)

    def wait_gather(b):
        pltpu.make_async_copy(table_hbm.at[pl.ds(0, CH)], rows[b], gsem).wait()

    def start_scatter(c_local, b):
        dst = out_hbm.at[pl.ds((chunk0 + c_local) * CH, CH)]
        pltpu.async_copy(rows[b], dst, osem[b])

    def wait_scatter(b):
        pltpu.make_async_copy(rows[b], out_hbm.at[pl.ds(0, CH)], osem[b]).wait()

    pltpu.sync_copy(ids_hbm.at[pl.ds(chunk0, CPW)], idx_v)
    for b in range(NBUF - 1):
        start_gather(b, b)
    pltpu.sync_copy(pos_hbm.at[pl.ds(0, S)], pos_v)

    def outer(g, carry):
        for b in range(NBUF):
            c = g * NBUF + b
            wait_gather(b)
            pbase = lax.rem(c * CH, S)

            def add_row(j, carry2, _b=b, _pbase=pbase):
                pj = lax.rem(_pbase + j, S)
                for k in range(D // 16):
                    sl = pl.ds(k * 16, 16)
                    rows[_b][j, sl] = rows[_b][j, sl] + pos_v[pj, sl]
                return carry2

            lax.fori_loop(0, CH, add_row, 0)
            start_scatter(c, b)
            nb = (b + NBUF - 1) % NBUF

            @pl.when(c + NBUF - 1 < CPW)
            def _(_c=c, _nb=nb):
                @pl.when(_c >= 1)
                def _():
                    wait_scatter(_nb)

                start_gather(_c + NBUF - 1, _nb)

        return carry

    lax.fori_loop(0, CPW // NBUF, outer, 0)
    for b in range(NBUF):
        wait_scatter(b)


def kernel(token_ids, token_table, pos_table):
    bsz, seq = token_ids.shape
    ids = token_ids.astype(jnp.int32).reshape(-1, CH)
    out = _sc_embed(ids, token_table, pos_table)
    return out.reshape(bsz, seq, token_table.shape[1])


# speedup vs baseline: 9.1029x; 3.0541x over previous
import functools

import jax
import jax.numpy as jnp
from jax import lax
from jax.experimental import pallas as pl
from jax.experimental.pallas import tpu as pltpu
from jax.experimental.pallas import tpu_sc as plsc

D = 128
S = 200
B = 4096
CH = 128
NBUF = 3
NC, NS = 2, 16
NW = NC * NS
NROWS = B * S
NCHUNK = NROWS // CH
CPW = NCHUNK // NW
MAIN = (CPW // NBUF) * NBUF

_mesh = plsc.VectorSubcoreMesh(core_axis_name="c", subcore_axis_name="s")


@functools.partial(
    pl.kernel,
    mesh=_mesh,
    out_type=jax.ShapeDtypeStruct((NROWS, D), jnp.float32),
    scratch_types=[
        pltpu.VMEM((CPW, CH), jnp.int32),
        pltpu.VMEM((CPW, CH), jnp.int32),
        pltpu.VMEM((S, D), jnp.float32),
        pltpu.VMEM((CH, D), jnp.float32),
        pltpu.VMEM((CH, D), jnp.float32),
        pltpu.VMEM((CH, D), jnp.float32),
        pltpu.SemaphoreType.DMA,
        pltpu.SemaphoreType.DMA,
        pltpu.SemaphoreType.DMA,
        pltpu.SemaphoreType.DMA,
    ],
)
def _sc_embed(ids_hbm, oidx_hbm, table_hbm, pos_hbm, out_hbm,
              idx_v, oidx_v, pos_v, r0, r1, r2, gsem, o0, o1, o2):
    rows = (r0, r1, r2)
    osem = (o0, o1, o2)
    wid = lax.axis_index("s") * NC + lax.axis_index("c")
    chunk0 = wid * CPW

    def start_gather(c_local, b):
        pltpu.async_copy(table_hbm.at[idx_v.at[c_local]], rows[b], gsem)

    def wait_gather(b):
        pltpu.make_async_copy(table_hbm.at[pl.ds(0, CH)], rows[b], gsem).wait()

    def start_scatter(c_local, b):
        pltpu.async_copy(rows[b], out_hbm.at[oidx_v.at[c_local]], osem[b])

    def wait_scatter(b):
        pltpu.make_async_copy(rows[b], out_hbm.at[pl.ds(0, CH)], osem[b]).wait()

    def add_pos(c, b):
        pv = [pos_v[c, pl.ds(k * 16, 16)] for k in range(D // 16)]

        def add_rows(j, carry2, _b=b, _pv=pv):
            for u in range(2):
                for k in range(D // 16):
                    sl = pl.ds(k * 16, 16)
                    jj = j * 2 + u
                    rows[_b][jj, sl] = rows[_b][jj, sl] + _pv[k]
            return carry2

        lax.fori_loop(0, CH // 2, add_rows, 0)

    pltpu.sync_copy(ids_hbm.at[pl.ds(chunk0, CPW)], idx_v)
    for b in range(NBUF - 1):
        start_gather(b, b)
    pltpu.sync_copy(oidx_hbm.at[pl.ds(chunk0, CPW)], oidx_v)
    pltpu.sync_copy(pos_hbm.at[pl.ds(0, S)], pos_v)

    def outer(g, carry):
        for b in range(NBUF):
            c = g * NBUF + b
            wait_gather(b)
            add_pos(c, b)
            start_scatter(c, b)
            nb = (b + NBUF - 1) % NBUF

            @pl.when(c + NBUF - 1 < CPW)
            def _(_c=c, _nb=nb):
                @pl.when(_c >= 1)
                def _():
                    wait_scatter(_nb)

                start_gather(_c + NBUF - 1, _nb)

        return carry

    lax.fori_loop(0, MAIN // NBUF, outer, 0)
    for c in range(MAIN, CPW):
        b = c % NBUF
        wait_gather(b)
        add_pos(c, b)
        start_scatter(c, b)
    for b in range(NBUF):
        wait_scatter(b)


def kernel(token_ids, token_table, pos_table):
    bsz, seq = token_ids.shape
    ids = (token_ids.astype(jnp.int32).T
           .reshape(S, NW, CH)
           .transpose(1, 0, 2)
           .reshape(NCHUNK, CH))
    bidx = (jnp.arange(NW, dtype=jnp.int32)[:, None, None] * CH
            + jnp.arange(CH, dtype=jnp.int32)[None, None, :])
    oidx = (bidx * S
            + jnp.arange(S, dtype=jnp.int32)[None, :, None]
            ).reshape(NCHUNK, CH)
    out = _sc_embed(ids, oidx, token_table, pos_table)
    return out.reshape(bsz, seq, token_table.shape[1])
